# baseline (speedup 1.0000x reference)
import functools

import jax
import jax.numpy as jnp
from jax import lax
from jax.experimental import pallas as pl
from jax.experimental.pallas import tpu as pltpu
from jax.experimental.pallas import tpu_sc as plsc

_N = 1_000_000
_C = 100
_N_BINS = 20
_ROW_BLOCK = 2000
_NW = 32
_LANES = 16
_CHUNK = 31264
_NPAD = _CHUNK * _NW
_KEYS = 64
_ITERS_FULL = _CHUNK // _LANES
_ITERS_LAST = (_N - (_NW - 1) * _CHUNK) // _LANES


def _dense_body(lowers_ref, logits_ref, labels_ref, conf_ref, key_ref):
    x = logits_ref[...]
    m = jnp.max(x, axis=1, keepdims=True)
    s = jnp.sum(jnp.exp(x - m), axis=1, keepdims=True)
    conf = 1.0 / s
    col = lax.broadcasted_iota(jnp.int32, x.shape, 1)
    pred = jnp.min(jnp.where(x == m, col, _C), axis=1, keepdims=True)
    acc = (pred == labels_ref[...]).astype(jnp.int32)
    low = lowers_ref[...]
    nbelow = jnp.sum((conf > low).astype(jnp.int32), axis=1, keepdims=True)
    bin_ = nbelow - 1
    conf_ref[...] = conf
    key_ref[...] = acc * 32 + bin_


def _sc_hist_body(key_hbm, conf_hbm, cnt_out, sconf_out,
                  key_v, conf_v, acc_cnt, acc_conf):
    wid = lax.axis_index("s") * 2 + lax.axis_index("c")
    base = wid * _CHUNK
    pltpu.sync_copy(key_hbm.at[pl.ds(base, _CHUNK)], key_v)
    pltpu.sync_copy(conf_hbm.at[pl.ds(base, _CHUNK)], conf_v)
    zeros16 = jnp.zeros((_LANES,), jnp.float32)
    for k in range(_KEYS):
        acc_cnt[pl.ds(k * _LANES, _LANES)] = zeros16
        acc_conf[pl.ds(k * _LANES, _LANES)] = zeros16
    lane = lax.iota(jnp.int32, _LANES)
    ones16 = jnp.ones((_LANES,), jnp.float32)
    niter = jnp.where(wid == _NW - 1, _ITERS_LAST, _ITERS_FULL)

    def body(i, carry):
        k16 = key_v[pl.ds(i * _LANES, _LANES)]
        c16 = conf_v[pl.ds(i * _LANES, _LANES)]
        fi = k16 * _LANES + lane
        plsc.addupdate_scatter(acc_cnt, [fi], ones16)
        plsc.addupdate_scatter(acc_conf, [fi], c16)
        return carry

    lax.fori_loop(0, niter, body, 0)
    pltpu.sync_copy(acc_cnt, cnt_out.at[wid])
    pltpu.sync_copy(acc_conf, sconf_out.at[wid])


def _combine_body(cnt_ref, sconf_ref, mbin_ref, macc_ref, ece_ref, ys_ref):
    cnt_tot = jnp.sum(cnt_ref[...], axis=0, keepdims=True)
    sconf_tot = jnp.sum(sconf_ref[...], axis=0, keepdims=True)
    mbin = mbin_ref[...]
    macc = macc_ref[...]
    count = jnp.dot(cnt_tot, mbin, preferred_element_type=jnp.float32)
    sum_acc = jnp.dot(cnt_tot, macc, preferred_element_type=jnp.float32)
    sum_conf = jnp.dot(sconf_tot, mbin, preferred_element_type=jnp.float32)
    lane32 = lax.broadcasted_iota(jnp.int32, (1, 32), 1)
    valid = (lane32 < _N_BINS) & (count > 0.0)
    safe = jnp.maximum(count, 1.0)
    acc_in = jnp.where(valid, sum_acc / safe, 0.0)
    conf_in = jnp.where(valid, sum_conf / safe, 0.0)
    prop = count / float(_N)
    per_bin = jnp.where(valid, jnp.abs(conf_in - acc_in) * prop, 0.0)
    ece_ref[...] = jnp.sum(per_bin, keepdims=True).reshape(1, 1)
    ys_ref[...] = acc_in


def kernel(logits, labels):
    n, c = logits.shape
    b = _ROW_BLOCK
    grid = n // b

    bb = jnp.linspace(0.0, 1.0, _N_BINS + 1).astype(jnp.float32)
    lowers = jnp.concatenate(
        [bb[:_N_BINS], jnp.full((128 - _N_BINS,), 2.0, jnp.float32)]
    ).reshape(1, 128)

    conf_p, key_p = pl.pallas_call(
        _dense_body,
        grid=(grid,),
        in_specs=[
            pl.BlockSpec((1, 128), lambda i: (0, 0)),
            pl.BlockSpec((b, c), lambda i: (i, 0)),
            pl.BlockSpec((b, 1), lambda i: (i, 0)),
        ],
        out_specs=[
            pl.BlockSpec((b, 1), lambda i: (i, 0)),
            pl.BlockSpec((b, 1), lambda i: (i, 0)),
        ],
        out_shape=[
            jax.ShapeDtypeStruct((_NPAD, 1), jnp.float32),
            jax.ShapeDtypeStruct((_NPAD, 1), jnp.int32),
        ],
    )(lowers, logits, labels.reshape(n, 1))

    mesh = plsc.VectorSubcoreMesh(
        core_axis_name="c", subcore_axis_name="s",
        num_cores=2, num_subcores=16,
    )
    sc_hist = pl.kernel(
        _sc_hist_body,
        out_type=[
            jax.ShapeDtypeStruct((_NW, _KEYS * _LANES), jnp.float32),
            jax.ShapeDtypeStruct((_NW, _KEYS * _LANES), jnp.float32),
        ],
        mesh=mesh,
        compiler_params=pltpu.CompilerParams(needs_layout_passes=False),
        scratch_types=[
            pltpu.VMEM((_CHUNK,), jnp.int32),
            pltpu.VMEM((_CHUNK,), jnp.float32),
            pltpu.VMEM((_KEYS * _LANES,), jnp.float32),
            pltpu.VMEM((_KEYS * _LANES,), jnp.float32),
        ],
    )
    cnt_part, sconf_part = sc_hist(key_p.reshape(_NPAD), conf_p.reshape(_NPAD))

    flat_key = jnp.arange(_KEYS * _LANES, dtype=jnp.int32) // _LANES
    bins = jnp.arange(32, dtype=jnp.int32)
    mbin = ((flat_key % 32)[:, None] == bins[None, :]).astype(jnp.float32)
    macc = (flat_key[:, None] == (bins[None, :] + 32)).astype(jnp.float32)

    ece2, ys2 = pl.pallas_call(
        _combine_body,
        in_specs=[
            pl.BlockSpec((_NW, _KEYS * _LANES), lambda: (0, 0)),
            pl.BlockSpec((_NW, _KEYS * _LANES), lambda: (0, 0)),
            pl.BlockSpec((_KEYS * _LANES, 32), lambda: (0, 0)),
            pl.BlockSpec((_KEYS * _LANES, 32), lambda: (0, 0)),
        ],
        out_specs=[
            pl.BlockSpec((1, 1), lambda: (0, 0)),
            pl.BlockSpec((1, 32), lambda: (0, 0)),
        ],
        out_shape=[
            jax.ShapeDtypeStruct((1, 1), jnp.float32),
            jax.ShapeDtypeStruct((1, 32), jnp.float32),
        ],
    )(cnt_part, sconf_part, mbin, macc)

    return (ece2.reshape(1), ys2[0, :_N_BINS])


# speedup vs baseline: 1.4863x; 1.4863x over previous
import numpy as np

import jax
import jax.numpy as jnp
from jax import lax
from jax.experimental import pallas as pl
from jax.experimental.pallas import tpu as pltpu
from jax.experimental.pallas import tpu_sc as plsc

_N = 1_000_000
_C = 100
_N_BINS = 20
_ROW_BLOCK = 2048
_GRID = 489
_NPAD = _GRID * _ROW_BLOCK
_NW = 32
_LANES = 16
_CHUNK = _NPAD // _NW
_ITERS = _CHUNK // _LANES
_KEYS = 64
_PAD_KEY = _KEYS - 1

_LOWERS = [float(v) for v in np.linspace(0.0, 1.0, _N_BINS + 1).astype(np.float32)[:_N_BINS]]


def _dense_body(logits_ref, labels_ref, conf_ref, key_ref):
    x = logits_ref[...]
    m = jnp.max(x, axis=1)
    s = jnp.sum(jnp.exp(x - m[:, None]), axis=1)
    conf = 1.0 / s
    col = lax.broadcasted_iota(jnp.int32, x.shape, 1)
    pred = jnp.min(jnp.where(x == m[:, None], col, _C), axis=1)
    acc = (pred == labels_ref[...]).astype(jnp.int32)
    row = pl.program_id(0) * _ROW_BLOCK + lax.broadcasted_iota(
        jnp.int32, (_ROW_BLOCK,), 0)
    valid = row < _N
    conf = jnp.where(valid, conf, 0.0)
    nbelow = jnp.zeros((_ROW_BLOCK,), jnp.int32)
    for lo in _LOWERS:
        nbelow = nbelow + (conf > lo).astype(jnp.int32)
    key = acc * 32 + (nbelow - 1)
    key_ref[...] = jnp.where(valid, key, _PAD_KEY)
    conf_ref[...] = conf


def _sc_hist_body(key_hbm, conf_hbm, cnt_out, sconf_out,
                  key_v, conf_v, acc_cnt, acc_conf):
    wid = lax.axis_index("s") * 2 + lax.axis_index("c")
    base = wid * _CHUNK
    pltpu.sync_copy(key_hbm.at[pl.ds(base, _CHUNK)], key_v)
    pltpu.sync_copy(conf_hbm.at[pl.ds(base, _CHUNK)], conf_v)
    zeros16 = jnp.zeros((_LANES,), jnp.float32)
    for k in range(_KEYS):
        acc_cnt[pl.ds(k * _LANES, _LANES)] = zeros16
        acc_conf[pl.ds(k * _LANES, _LANES)] = zeros16
    lane = lax.iota(jnp.int32, _LANES)
    ones16 = jnp.ones((_LANES,), jnp.float32)

    def body(i, carry):
        k16 = key_v[pl.ds(i * _LANES, _LANES)]
        c16 = conf_v[pl.ds(i * _LANES, _LANES)]
        fi = k16 * _LANES + lane
        plsc.addupdate_scatter(acc_cnt, [fi], ones16)
        plsc.addupdate_scatter(acc_conf, [fi], c16)
        return carry

    lax.fori_loop(0, _ITERS, body, 0)
    pltpu.sync_copy(acc_cnt, cnt_out.at[wid])
    pltpu.sync_copy(acc_conf, sconf_out.at[wid])


def _combine_body(cnt_ref, sconf_ref, mbin_ref, macc_ref, ece_ref, ys_ref):
    cnt_tot = jnp.sum(cnt_ref[...], axis=0, keepdims=True)
    sconf_tot = jnp.sum(sconf_ref[...], axis=0, keepdims=True)
    mbin = mbin_ref[...]
    macc = macc_ref[...]
    count = jnp.dot(cnt_tot, mbin, preferred_element_type=jnp.float32)
    sum_acc = jnp.dot(cnt_tot, macc, preferred_element_type=jnp.float32)
    sum_conf = jnp.dot(sconf_tot, mbin, preferred_element_type=jnp.float32)
    lane32 = lax.broadcasted_iota(jnp.int32, (1, 32), 1)
    valid = (lane32 < _N_BINS) & (count > 0.0)
    safe = jnp.maximum(count, 1.0)
    acc_in = jnp.where(valid, sum_acc / safe, 0.0)
    conf_in = jnp.where(valid, sum_conf / safe, 0.0)
    prop = count / float(_N)
    per_bin = jnp.where(valid, jnp.abs(conf_in - acc_in) * prop, 0.0)
    ece_ref[...] = jnp.sum(per_bin, keepdims=True).reshape(1, 1)
    ys_ref[...] = acc_in


def kernel(logits, labels):
    n, c = logits.shape
    b = _ROW_BLOCK

    conf_p, key_p = pl.pallas_call(
        _dense_body,
        grid=(_GRID,),
        in_specs=[
            pl.BlockSpec((b, c), lambda i: (i, 0)),
            pl.BlockSpec((b,), lambda i: (i,)),
        ],
        out_specs=[
            pl.BlockSpec((b,), lambda i: (i,)),
            pl.BlockSpec((b,), lambda i: (i,)),
        ],
        out_shape=[
            jax.ShapeDtypeStruct((_NPAD,), jnp.float32),
            jax.ShapeDtypeStruct((_NPAD,), jnp.int32),
        ],
    )(logits, labels)

    mesh = plsc.VectorSubcoreMesh(
        core_axis_name="c", subcore_axis_name="s",
        num_cores=2, num_subcores=16,
    )
    sc_hist = pl.kernel(
        _sc_hist_body,
        out_type=[
            jax.ShapeDtypeStruct((_NW, _KEYS * _LANES), jnp.float32),
            jax.ShapeDtypeStruct((_NW, _KEYS * _LANES), jnp.float32),
        ],
        mesh=mesh,
        compiler_params=pltpu.CompilerParams(needs_layout_passes=False),
        scratch_types=[
            pltpu.VMEM((_CHUNK,), jnp.int32),
            pltpu.VMEM((_CHUNK,), jnp.float32),
            pltpu.VMEM((_KEYS * _LANES,), jnp.float32),
            pltpu.VMEM((_KEYS * _LANES,), jnp.float32),
        ],
    )
    cnt_part, sconf_part = sc_hist(key_p, conf_p)

    flat_key = jnp.arange(_KEYS * _LANES, dtype=jnp.int32) // _LANES
    bins = jnp.arange(32, dtype=jnp.int32)
    mbin = ((flat_key % 32)[:, None] == bins[None, :]).astype(jnp.float32)
    macc = (flat_key[:, None] == (bins[None, :] + 32)).astype(jnp.float32)

    ece2, ys2 = pl.pallas_call(
        _combine_body,
        in_specs=[
            pl.BlockSpec((_NW, _KEYS * _LANES), lambda: (0, 0)),
            pl.BlockSpec((_NW, _KEYS * _LANES), lambda: (0, 0)),
            pl.BlockSpec((_KEYS * _LANES, 32), lambda: (0, 0)),
            pl.BlockSpec((_KEYS * _LANES, 32), lambda: (0, 0)),
        ],
        out_specs=[
            pl.BlockSpec((1, 1), lambda: (0, 0)),
            pl.BlockSpec((1, 32), lambda: (0, 0)),
        ],
        out_shape=[
            jax.ShapeDtypeStruct((1, 1), jnp.float32),
            jax.ShapeDtypeStruct((1, 32), jnp.float32),
        ],
    )(cnt_part, sconf_part, mbin, macc)

    return (ece2.reshape(1), ys2[0, :_N_BINS])


# speedup vs baseline: 1.6368x; 1.1013x over previous
import jax
import jax.numpy as jnp
from jax import lax
from jax.experimental import pallas as pl
from jax.experimental.pallas import tpu as pltpu
from jax.experimental.pallas import tpu_sc as plsc

_N = 1_000_000
_C = 100
_N_BINS = 20
_ROW_BLOCK = 8192
_GRID = 123
_NPAD = _GRID * _ROW_BLOCK
_NW = 32
_LANES = 16
_CHUNK = _NPAD // _NW
_ITERS = _CHUNK // _LANES
_KEYS = 64


def _dense_body(logits_ref, labels_ref, v_ref):
    x = logits_ref[...]
    m = jnp.max(x, axis=1, keepdims=True)
    s = jnp.sum(jnp.exp(x - m), axis=1, keepdims=True)
    col = lax.broadcasted_iota(jnp.int32, x.shape, 1)
    xl = jnp.max(jnp.where(col == labels_ref[...][:, None], x, -jnp.inf),
                 axis=1, keepdims=True)
    v = jnp.where(xl == m, -s, s)
    v_ref[...] = v.reshape(_ROW_BLOCK)


def _sc_hist_body(v_hbm, cnt_out, sconf_out, v_v, acc_cnt, acc_conf):
    wid = lax.axis_index("s") * 2 + lax.axis_index("c")
    base = wid * _CHUNK
    pltpu.sync_copy(v_hbm.at[pl.ds(base, _CHUNK)], v_v)
    zeros16 = jnp.zeros((_LANES,), jnp.float32)
    for k in range(_KEYS):
        acc_cnt[pl.ds(k * _LANES, _LANES)] = zeros16
        acc_conf[pl.ds(k * _LANES, _LANES)] = zeros16
    lane = lax.iota(jnp.int32, _LANES)
    ones16 = jnp.ones((_LANES,), jnp.float32)

    def body(i, gidx):
        v16 = v_v[pl.ds(i * _LANES, _LANES)]
        conf = 1.0 / jnp.abs(v16)
        bin_ = lax.convert_element_type(conf * float(_N_BINS), jnp.int32)
        bin_ = jnp.minimum(jnp.maximum(bin_, 0), _N_BINS - 1)
        fi = jnp.where(v16 < 0.0, 32 * _LANES, 0) + bin_ * _LANES + lane
        mask = gidx < _N
        plsc.addupdate_scatter(acc_cnt, [fi], ones16, mask=mask)
        plsc.addupdate_scatter(acc_conf, [fi], conf, mask=mask)
        return gidx + _LANES

    lax.fori_loop(0, _ITERS, body, base + lane)
    pltpu.sync_copy(acc_cnt, cnt_out.at[wid])
    pltpu.sync_copy(acc_conf, sconf_out.at[wid])


def _combine_body(cnt_ref, sconf_ref, mbin_ref, macc_ref, ece_ref, ys_ref):
    cnt_tot = jnp.sum(cnt_ref[...], axis=0, keepdims=True)
    sconf_tot = jnp.sum(sconf_ref[...], axis=0, keepdims=True)
    mbin = mbin_ref[...]
    macc = macc_ref[...]
    count = jnp.dot(cnt_tot, mbin, preferred_element_type=jnp.float32)
    sum_acc = jnp.dot(cnt_tot, macc, preferred_element_type=jnp.float32)
    sum_conf = jnp.dot(sconf_tot, mbin, preferred_element_type=jnp.float32)
    lane32 = lax.broadcasted_iota(jnp.int32, (1, 32), 1)
    valid = (lane32 < _N_BINS) & (count > 0.0)
    safe = jnp.maximum(count, 1.0)
    acc_in = jnp.where(valid, sum_acc / safe, 0.0)
    conf_in = jnp.where(valid, sum_conf / safe, 0.0)
    prop = count / float(_N)
    per_bin = jnp.where(valid, jnp.abs(conf_in - acc_in) * prop, 0.0)
    ece_ref[...] = jnp.sum(per_bin, keepdims=True).reshape(1, 1)
    ys_ref[...] = acc_in


def kernel(logits, labels):
    n, c = logits.shape
    b = _ROW_BLOCK

    v_p = pl.pallas_call(
        _dense_body,
        grid=(_GRID,),
        in_specs=[
            pl.BlockSpec((b, c), lambda i: (i, 0)),
            pl.BlockSpec((b,), lambda i: (i,)),
        ],
        out_specs=pl.BlockSpec((b,), lambda i: (i,)),
        out_shape=jax.ShapeDtypeStruct((_NPAD,), jnp.float32),
    )(logits, labels)

    mesh = plsc.VectorSubcoreMesh(
        core_axis_name="c", subcore_axis_name="s",
        num_cores=2, num_subcores=16,
    )
    sc_hist = pl.kernel(
        _sc_hist_body,
        out_type=[
            jax.ShapeDtypeStruct((_NW, _KEYS * _LANES), jnp.float32),
            jax.ShapeDtypeStruct((_NW, _KEYS * _LANES), jnp.float32),
        ],
        mesh=mesh,
        compiler_params=pltpu.CompilerParams(needs_layout_passes=False),
        scratch_types=[
            pltpu.VMEM((_CHUNK,), jnp.float32),
            pltpu.VMEM((_KEYS * _LANES,), jnp.float32),
            pltpu.VMEM((_KEYS * _LANES,), jnp.float32),
        ],
    )
    cnt_part, sconf_part = sc_hist(v_p)

    flat_key = jnp.arange(_KEYS * _LANES, dtype=jnp.int32) // _LANES
    bins = jnp.arange(32, dtype=jnp.int32)
    mbin = ((flat_key % 32)[:, None] == bins[None, :]).astype(jnp.float32)
    macc = (flat_key[:, None] == (bins[None, :] + 32)).astype(jnp.float32)

    ece2, ys2 = pl.pallas_call(
        _combine_body,
        in_specs=[
            pl.BlockSpec((_NW, _KEYS * _LANES), lambda: (0, 0)),
            pl.BlockSpec((_NW, _KEYS * _LANES), lambda: (0, 0)),
            pl.BlockSpec((_KEYS * _LANES, 32), lambda: (0, 0)),
            pl.BlockSpec((_KEYS * _LANES, 32), lambda: (0, 0)),
        ],
        out_specs=[
            pl.BlockSpec((1, 1), lambda: (0, 0)),
            pl.BlockSpec((1, 32), lambda: (0, 0)),
        ],
        out_shape=[
            jax.ShapeDtypeStruct((1, 1), jnp.float32),
            jax.ShapeDtypeStruct((1, 32), jnp.float32),
        ],
    )(cnt_part, sconf_part, mbin, macc)

    return (ece2.reshape(1), ys2[0, :_N_BINS])


# speedup vs baseline: 1.7717x; 1.0824x over previous
import jax
import jax.numpy as jnp
from jax import lax
from jax.experimental import pallas as pl
from jax.experimental.pallas import tpu as pltpu
from jax.experimental.pallas import tpu_sc as plsc

_N = 1_000_000
_C = 100
_N_BINS = 20
_ROW_BLOCK = 8192
_GRID = 123
_NPAD = _GRID * _ROW_BLOCK
_NW = 32
_LANES = 16
_CHUNK = _NPAD // _NW
_ITERS = _CHUNK // _LANES
_KEYS = 64


def _dense_body(logits_ref, labels_ref, v_ref):
    x = logits_ref[...]
    col = lax.broadcasted_iota(jnp.int32, x.shape, 1)
    xp = lax.bitcast_convert_type(
        (lax.bitcast_convert_type(x, jnp.int32) & ~0x7F) | col, jnp.float32)
    mm = jnp.max(xp, axis=1, keepdims=True)
    s = jnp.sum(jnp.exp(x - mm), axis=1, keepdims=True)
    mm1 = mm.reshape(_ROW_BLOCK)
    s1 = s.reshape(_ROW_BLOCK)
    pred = lax.bitcast_convert_type(mm1, jnp.int32) & 0x7F
    v_ref[...] = jnp.where(pred == labels_ref[...], -s1, s1)


def _sc_hist_body(v_hbm, cnt_out, sconf_out, v_v, acc_cnt, acc_conf):
    wid = lax.axis_index("s") * 2 + lax.axis_index("c")
    base = wid * _CHUNK
    pltpu.sync_copy(v_hbm.at[pl.ds(base, _CHUNK)], v_v)
    zeros16 = jnp.zeros((_LANES,), jnp.float32)
    for k in range(_KEYS):
        acc_cnt[pl.ds(k * _LANES, _LANES)] = zeros16
        acc_conf[pl.ds(k * _LANES, _LANES)] = zeros16
    lane = lax.iota(jnp.int32, _LANES)
    ones16 = jnp.ones((_LANES,), jnp.float32)

    def body(i, gidx):
        v16 = v_v[pl.ds(i * _LANES, _LANES)]
        conf = 1.0 / jnp.abs(v16)
        bin_ = lax.convert_element_type(conf * float(_N_BINS), jnp.int32)
        bin_ = jnp.minimum(jnp.maximum(bin_, 0), _N_BINS - 1)
        fi = jnp.where(v16 < 0.0, 32 * _LANES, 0) + bin_ * _LANES + lane
        mask = gidx < _N
        plsc.addupdate_scatter(acc_cnt, [fi], ones16, mask=mask)
        plsc.addupdate_scatter(acc_conf, [fi], conf, mask=mask)
        return gidx + _LANES

    lax.fori_loop(0, _ITERS, body, base + lane)
    pltpu.sync_copy(acc_cnt, cnt_out.at[wid])
    pltpu.sync_copy(acc_conf, sconf_out.at[wid])


def _combine_body(cnt_ref, sconf_ref, mbin_ref, macc_ref, ece_ref, ys_ref):
    cnt_tot = jnp.sum(cnt_ref[...], axis=0, keepdims=True)
    sconf_tot = jnp.sum(sconf_ref[...], axis=0, keepdims=True)
    mbin = mbin_ref[...]
    macc = macc_ref[...]
    count = jnp.dot(cnt_tot, mbin, preferred_element_type=jnp.float32)
    sum_acc = jnp.dot(cnt_tot, macc, preferred_element_type=jnp.float32)
    sum_conf = jnp.dot(sconf_tot, mbin, preferred_element_type=jnp.float32)
    lane32 = lax.broadcasted_iota(jnp.int32, (1, 32), 1)
    valid = (lane32 < _N_BINS) & (count > 0.0)
    safe = jnp.maximum(count, 1.0)
    acc_in = jnp.where(valid, sum_acc / safe, 0.0)
    conf_in = jnp.where(valid, sum_conf / safe, 0.0)
    prop = count / float(_N)
    per_bin = jnp.where(valid, jnp.abs(conf_in - acc_in) * prop, 0.0)
    ece_ref[...] = jnp.sum(per_bin, keepdims=True).reshape(1, 1)
    ys_ref[...] = acc_in


def kernel(logits, labels):
    n, c = logits.shape
    b = _ROW_BLOCK

    v_p = pl.pallas_call(
        _dense_body,
        grid=(_GRID,),
        in_specs=[
            pl.BlockSpec((b, c), lambda i: (i, 0)),
            pl.BlockSpec((b,), lambda i: (i,)),
        ],
        out_specs=pl.BlockSpec((b,), lambda i: (i,)),
        out_shape=jax.ShapeDtypeStruct((_NPAD,), jnp.float32),
    )(logits, labels)

    mesh = plsc.VectorSubcoreMesh(
        core_axis_name="c", subcore_axis_name="s",
        num_cores=2, num_subcores=16,
    )
    sc_hist = pl.kernel(
        _sc_hist_body,
        out_type=[
            jax.ShapeDtypeStruct((_NW, _KEYS * _LANES), jnp.float32),
            jax.ShapeDtypeStruct((_NW, _KEYS * _LANES), jnp.float32),
        ],
        mesh=mesh,
        compiler_params=pltpu.CompilerParams(needs_layout_passes=False),
        scratch_types=[
            pltpu.VMEM((_CHUNK,), jnp.float32),
            pltpu.VMEM((_KEYS * _LANES,), jnp.float32),
            pltpu.VMEM((_KEYS * _LANES,), jnp.float32),
        ],
    )
    cnt_part, sconf_part = sc_hist(v_p)

    flat_key = jnp.arange(_KEYS * _LANES, dtype=jnp.int32) // _LANES
    bins = jnp.arange(32, dtype=jnp.int32)
    mbin = ((flat_key % 32)[:, None] == bins[None, :]).astype(jnp.float32)
    macc = (flat_key[:, None] == (bins[None, :] + 32)).astype(jnp.float32)

    ece2, ys2 = pl.pallas_call(
        _combine_body,
        in_specs=[
            pl.BlockSpec((_NW, _KEYS * _LANES), lambda: (0, 0)),
            pl.BlockSpec((_NW, _KEYS * _LANES), lambda: (0, 0)),
            pl.BlockSpec((_KEYS * _LANES, 32), lambda: (0, 0)),
            pl.BlockSpec((_KEYS * _LANES, 32), lambda: (0, 0)),
        ],
        out_specs=[
            pl.BlockSpec((1, 1), lambda: (0, 0)),
            pl.BlockSpec((1, 32), lambda: (0, 0)),
        ],
        out_shape=[
            jax.ShapeDtypeStruct((1, 1), jnp.float32),
            jax.ShapeDtypeStruct((1, 32), jnp.float32),
        ],
    )(cnt_part, sconf_part, mbin, macc)

    return (ece2.reshape(1), ys2[0, :_N_BINS])


# speedup vs baseline: 1.7731x; 1.0008x over previous
import jax
import jax.numpy as jnp
from jax import lax
from jax.experimental import pallas as pl
from jax.experimental.pallas import tpu as pltpu
from jax.experimental.pallas import tpu_sc as plsc

_N = 1_000_000
_C = 100
_N_BINS = 20
_ROW_BLOCK = 16384
_GRID = 62
_NPAD = _GRID * _ROW_BLOCK
_NW = 32
_LANES = 16
_CHUNK = _NPAD // _NW
_ITERS = _CHUNK // _LANES
_KEYS = 64


def _dense_body(logits_ref, labels_ref, v_ref):
    x = logits_ref[...]
    col = lax.broadcasted_iota(jnp.int32, x.shape, 1)
    xp = lax.bitcast_convert_type(
        (lax.bitcast_convert_type(x, jnp.int32) & ~0x7F) | col, jnp.float32)
    mm = jnp.max(xp, axis=1, keepdims=True)
    s = jnp.sum(jnp.exp(x - mm), axis=1, keepdims=True)
    mm1 = mm.reshape(_ROW_BLOCK)
    s1 = s.reshape(_ROW_BLOCK)
    pred = lax.bitcast_convert_type(mm1, jnp.int32) & 0x7F
    v_ref[...] = jnp.where(pred == labels_ref[...], -s1, s1)


def _sc_hist_body(v_hbm, cnt_out, sconf_out, v_v, acc_cnt, acc_conf):
    wid = lax.axis_index("s") * 2 + lax.axis_index("c")
    base = wid * _CHUNK
    pltpu.sync_copy(v_hbm.at[pl.ds(base, _CHUNK)], v_v)
    zeros16 = jnp.zeros((_LANES,), jnp.float32)
    for k in range(_KEYS):
        acc_cnt[pl.ds(k * _LANES, _LANES)] = zeros16
        acc_conf[pl.ds(k * _LANES, _LANES)] = zeros16
    lane = lax.iota(jnp.int32, _LANES)
    ones16 = jnp.ones((_LANES,), jnp.float32)

    def body(i, gidx):
        v16 = v_v[pl.ds(i * _LANES, _LANES)]
        conf = 1.0 / jnp.abs(v16)
        bin_ = lax.convert_element_type(conf * float(_N_BINS), jnp.int32)
        bin_ = jnp.minimum(jnp.maximum(bin_, 0), _N_BINS - 1)
        fi = jnp.where(v16 < 0.0, 32 * _LANES, 0) + bin_ * _LANES + lane
        mask = gidx < _N
        plsc.addupdate_scatter(acc_cnt, [fi], ones16, mask=mask)
        plsc.addupdate_scatter(acc_conf, [fi], conf, mask=mask)
        return gidx + _LANES

    lax.fori_loop(0, _ITERS, body, base + lane)
    pltpu.sync_copy(acc_cnt, cnt_out.at[wid])
    pltpu.sync_copy(acc_conf, sconf_out.at[wid])


def _combine_body(cnt_ref, sconf_ref, mbin_ref, macc_ref, ece_ref, ys_ref):
    cnt_tot = jnp.sum(cnt_ref[...], axis=0, keepdims=True)
    sconf_tot = jnp.sum(sconf_ref[...], axis=0, keepdims=True)
    mbin = mbin_ref[...]
    macc = macc_ref[...]
    count = jnp.dot(cnt_tot, mbin, preferred_element_type=jnp.float32)
    sum_acc = jnp.dot(cnt_tot, macc, preferred_element_type=jnp.float32)
    sum_conf = jnp.dot(sconf_tot, mbin, preferred_element_type=jnp.float32)
    lane32 = lax.broadcasted_iota(jnp.int32, (1, 32), 1)
    valid = (lane32 < _N_BINS) & (count > 0.0)
    safe = jnp.maximum(count, 1.0)
    acc_in = jnp.where(valid, sum_acc / safe, 0.0)
    conf_in = jnp.where(valid, sum_conf / safe, 0.0)
    prop = count / float(_N)
    per_bin = jnp.where(valid, jnp.abs(conf_in - acc_in) * prop, 0.0)
    ece_ref[...] = jnp.sum(per_bin, keepdims=True).reshape(1, 1)
    ys_ref[...] = acc_in


def kernel(logits, labels):
    n, c = logits.shape
    b = _ROW_BLOCK

    v_p = pl.pallas_call(
        _dense_body,
        grid=(_GRID,),
        in_specs=[
            pl.BlockSpec((b, c), lambda i: (i, 0)),
            pl.BlockSpec((b,), lambda i: (i,)),
        ],
        out_specs=pl.BlockSpec((b,), lambda i: (i,)),
        out_shape=jax.ShapeDtypeStruct((_NPAD,), jnp.float32),
    )(logits, labels)

    mesh = plsc.VectorSubcoreMesh(
        core_axis_name="c", subcore_axis_name="s",
        num_cores=2, num_subcores=16,
    )
    sc_hist = pl.kernel(
        _sc_hist_body,
        out_type=[
            jax.ShapeDtypeStruct((_NW, _KEYS * _LANES), jnp.float32),
            jax.ShapeDtypeStruct((_NW, _KEYS * _LANES), jnp.float32),
        ],
        mesh=mesh,
        compiler_params=pltpu.CompilerParams(needs_layout_passes=False),
        scratch_types=[
            pltpu.VMEM((_CHUNK,), jnp.float32),
            pltpu.VMEM((_KEYS * _LANES,), jnp.float32),
            pltpu.VMEM((_KEYS * _LANES,), jnp.float32),
        ],
    )
    cnt_part, sconf_part = sc_hist(v_p)

    flat_key = jnp.arange(_KEYS * _LANES, dtype=jnp.int32) // _LANES
    bins = jnp.arange(32, dtype=jnp.int32)
    mbin = ((flat_key % 32)[:, None] == bins[None, :]).astype(jnp.float32)
    macc = (flat_key[:, None] == (bins[None, :] + 32)).astype(jnp.float32)

    ece2, ys2 = pl.pallas_call(
        _combine_body,
        in_specs=[
            pl.BlockSpec((_NW, _KEYS * _LANES), lambda: (0, 0)),
            pl.BlockSpec((_NW, _KEYS * _LANES), lambda: (0, 0)),
            pl.BlockSpec((_KEYS * _LANES, 32), lambda: (0, 0)),
            pl.BlockSpec((_KEYS * _LANES, 32), lambda: (0, 0)),
        ],
        out_specs=[
            pl.BlockSpec((1, 1), lambda: (0, 0)),
            pl.BlockSpec((1, 32), lambda: (0, 0)),
        ],
        out_shape=[
            jax.ShapeDtypeStruct((1, 1), jnp.float32),
            jax.ShapeDtypeStruct((1, 32), jnp.float32),
        ],
    )(cnt_part, sconf_part, mbin, macc)

    return (ece2.reshape(1), ys2[0, :_N_BINS])


# speedup vs baseline: 2.2015x; 1.2416x over previous
import jax
import jax.numpy as jnp
from jax import lax
from jax.experimental import pallas as pl
from jax.experimental.pallas import tpu as pltpu
from jax.experimental.pallas import tpu_sc as plsc

_N = 1_000_000
_C = 100
_N_BINS = 20
_ROW_BLOCK = 8192
_GRID = 123
_NPAD = _GRID * _ROW_BLOCK
_NW = 32
_LANES = 16
_CHUNK = _NPAD // _NW
_ITERS = _CHUNK // _LANES
_KEYS = 64
_LAB_LAST = _N - (_NW - 1) * _CHUNK


def _dense_body(logits_ref, v_ref):
    x = logits_ref[...]
    col = lax.broadcasted_iota(jnp.int32, x.shape, 1)
    xp = lax.bitcast_convert_type(
        (lax.bitcast_convert_type(x, jnp.int32) & ~0x7F) | col, jnp.float32)
    mm = jnp.max(xp, axis=1, keepdims=True)
    s = jnp.sum(jnp.exp(xp - mm), axis=1, keepdims=True)
    si = lax.bitcast_convert_type(s, jnp.int32)
    mi = lax.bitcast_convert_type(mm, jnp.int32)
    v_ref[...] = lax.bitcast_convert_type((si & ~0x7F) | (mi & 0x7F),
                                          jnp.float32)


def _sc_hist_body(v_hbm, labels_hbm, cnt_out, sconf_out,
                  v_v, lab_v, acc_cnt, acc_conf):
    wid = lax.axis_index("s") * 2 + lax.axis_index("c")
    base = wid * _CHUNK
    pltpu.sync_copy(v_hbm.at[pl.ds(base, _CHUNK)], v_v)

    @pl.when(wid < _NW - 1)
    def _():
        pltpu.sync_copy(labels_hbm.at[pl.ds(base, _CHUNK)], lab_v)

    @pl.when(wid == _NW - 1)
    def _():
        pltpu.sync_copy(labels_hbm.at[pl.ds(base, _LAB_LAST)],
                        lab_v.at[pl.ds(0, _LAB_LAST)])

    zeros16 = jnp.zeros((_LANES,), jnp.float32)
    for k in range(_KEYS):
        acc_cnt[pl.ds(k * _LANES, _LANES)] = zeros16
        acc_conf[pl.ds(k * _LANES, _LANES)] = zeros16
    lane = lax.iota(jnp.int32, _LANES)
    ones16 = jnp.ones((_LANES,), jnp.float32)

    def body(i, gidx):
        v16 = v_v[pl.ds(i * _LANES, _LANES)]
        lab16 = lab_v[pl.ds(i * _LANES, _LANES)]
        pred = lax.bitcast_convert_type(v16, jnp.int32) & 0x7F
        conf = 1.0 / v16
        bin_ = lax.convert_element_type(conf * float(_N_BINS), jnp.int32)
        bin_ = jnp.minimum(jnp.maximum(bin_, 0), _N_BINS - 1)
        fi = jnp.where(pred == lab16, 32 * _LANES, 0) + bin_ * _LANES + lane
        mask = gidx < _N
        plsc.addupdate_scatter(acc_cnt, [fi], ones16, mask=mask)
        plsc.addupdate_scatter(acc_conf, [fi], conf, mask=mask)
        return gidx + _LANES

    lax.fori_loop(0, _ITERS, body, base + lane)
    pltpu.sync_copy(acc_cnt, cnt_out.at[wid])
    pltpu.sync_copy(acc_conf, sconf_out.at[wid])


def _combine_body(cnt_ref, sconf_ref, mbin_ref, macc_ref, ece_ref, ys_ref):
    cnt_tot = jnp.sum(cnt_ref[...], axis=0, keepdims=True)
    sconf_tot = jnp.sum(sconf_ref[...], axis=0, keepdims=True)
    mbin = mbin_ref[...]
    macc = macc_ref[...]
    count = jnp.dot(cnt_tot, mbin, preferred_element_type=jnp.float32)
    sum_acc = jnp.dot(cnt_tot, macc, preferred_element_type=jnp.float32)
    sum_conf = jnp.dot(sconf_tot, mbin, preferred_element_type=jnp.float32)
    lane32 = lax.broadcasted_iota(jnp.int32, (1, 32), 1)
    valid = (lane32 < _N_BINS) & (count > 0.0)
    safe = jnp.maximum(count, 1.0)
    acc_in = jnp.where(valid, sum_acc / safe, 0.0)
    conf_in = jnp.where(valid, sum_conf / safe, 0.0)
    prop = count / float(_N)
    per_bin = jnp.where(valid, jnp.abs(conf_in - acc_in) * prop, 0.0)
    ece_ref[...] = jnp.sum(per_bin, keepdims=True).reshape(1, 1)
    ys_ref[...] = acc_in


def kernel(logits, labels):
    n, c = logits.shape
    b = _ROW_BLOCK

    v_p = pl.pallas_call(
        _dense_body,
        grid=(_GRID,),
        in_specs=[pl.BlockSpec((b, c), lambda i: (i, 0))],
        out_specs=pl.BlockSpec((b, 1), lambda i: (i, 0)),
        out_shape=jax.ShapeDtypeStruct((_NPAD, 1), jnp.float32),
    )(logits)

    mesh = plsc.VectorSubcoreMesh(
        core_axis_name="c", subcore_axis_name="s",
        num_cores=2, num_subcores=16,
    )
    sc_hist = pl.kernel(
        _sc_hist_body,
        out_type=[
            jax.ShapeDtypeStruct((_NW, _KEYS * _LANES), jnp.float32),
            jax.ShapeDtypeStruct((_NW, _KEYS * _LANES), jnp.float32),
        ],
        mesh=mesh,
        compiler_params=pltpu.CompilerParams(needs_layout_passes=False),
        scratch_types=[
            pltpu.VMEM((_CHUNK,), jnp.float32),
            pltpu.VMEM((_CHUNK,), jnp.int32),
            pltpu.VMEM((_KEYS * _LANES,), jnp.float32),
            pltpu.VMEM((_KEYS * _LANES,), jnp.float32),
        ],
    )
    cnt_part, sconf_part = sc_hist(v_p.reshape(_NPAD), labels)

    flat_key = jnp.arange(_KEYS * _LANES, dtype=jnp.int32) // _LANES
    bins = jnp.arange(32, dtype=jnp.int32)
    mbin = ((flat_key % 32)[:, None] == bins[None, :]).astype(jnp.float32)
    macc = (flat_key[:, None] == (bins[None, :] + 32)).astype(jnp.float32)

    ece2, ys2 = pl.pallas_call(
        _combine_body,
        in_specs=[
            pl.BlockSpec((_NW, _KEYS * _LANES), lambda: (0, 0)),
            pl.BlockSpec((_NW, _KEYS * _LANES), lambda: (0, 0)),
            pl.BlockSpec((_KEYS * _LANES, 32), lambda: (0, 0)),
            pl.BlockSpec((_KEYS * _LANES, 32), lambda: (0, 0)),
        ],
        out_specs=[
            pl.BlockSpec((1, 1), lambda: (0, 0)),
            pl.BlockSpec((1, 32), lambda: (0, 0)),
        ],
        out_shape=[
            jax.ShapeDtypeStruct((1, 1), jnp.float32),
            jax.ShapeDtypeStruct((1, 32), jnp.float32),
        ],
    )(cnt_part, sconf_part, mbin, macc)

    return (ece2.reshape(1), ys2[0, :_N_BINS])


# speedup vs baseline: 2.7094x; 1.2307x over previous
import jax
import jax.numpy as jnp
from jax import lax
from jax.experimental import pallas as pl
from jax.experimental.pallas import tpu as pltpu
from jax.experimental.pallas import tpu_sc as plsc

_N = 1_000_000
_C = 100
_N_BINS = 20
_ROW_BLOCK = 8192
_GRID = 123
_NPAD = _GRID * _ROW_BLOCK
_NW = 32
_LANES = 16
_CHUNK = _NPAD // _NW
_ITERS = _CHUNK // _LANES
_KEYS = 64
_LAB_LAST = _N - (_NW - 1) * _CHUNK


def _dense_body(logits_ref, v_ref):
    x = logits_ref[...]
    col = lax.broadcasted_iota(jnp.int32, x.shape, 1)
    xp = lax.bitcast_convert_type(
        (lax.bitcast_convert_type(x, jnp.int32) & ~0x7F) | col, jnp.float32)
    mm = jnp.max(xp, axis=1, keepdims=True)
    s = jnp.sum(jnp.exp(xp - mm), axis=1, keepdims=True)
    si = lax.bitcast_convert_type(s, jnp.int32)
    mi = lax.bitcast_convert_type(mm, jnp.int32)
    v = lax.bitcast_convert_type((si & ~0x7F) | (mi & 0x7F), jnp.float32)
    v_ref[...] = v.reshape(_ROW_BLOCK // 8, 8)


def _sc_hist_body(v_hbm, labels_hbm, cnt_out, sconf_out,
                  v_v, lab_v, acc_cnt, acc_conf):
    wid = lax.axis_index("s") * 2 + lax.axis_index("c")
    base = wid * _CHUNK
    pltpu.sync_copy(v_hbm.at[pl.ds(base, _CHUNK)], v_v)

    @pl.when(wid < _NW - 1)
    def _():
        pltpu.sync_copy(labels_hbm.at[pl.ds(base, _CHUNK)], lab_v)

    @pl.when(wid == _NW - 1)
    def _():
        pltpu.sync_copy(labels_hbm.at[pl.ds(base, _LAB_LAST)],
                        lab_v.at[pl.ds(0, _LAB_LAST)])

    zeros16 = jnp.zeros((_LANES,), jnp.float32)
    for k in range(_KEYS):
        acc_cnt[pl.ds(k * _LANES, _LANES)] = zeros16
        acc_conf[pl.ds(k * _LANES, _LANES)] = zeros16
    lane = lax.iota(jnp.int32, _LANES)
    ones16 = jnp.ones((_LANES,), jnp.float32)

    def body(i, gidx):
        v16 = v_v[pl.ds(i * _LANES, _LANES)]
        lab16 = lab_v[pl.ds(i * _LANES, _LANES)]
        pred = lax.bitcast_convert_type(v16, jnp.int32) & 0x7F
        conf = 1.0 / v16
        bin_ = lax.convert_element_type(conf * float(_N_BINS), jnp.int32)
        bin_ = jnp.minimum(jnp.maximum(bin_, 0), _N_BINS - 1)
        fi = jnp.where(pred == lab16, 32 * _LANES, 0) + bin_ * _LANES + lane
        mask = gidx < _N
        plsc.addupdate_scatter(acc_cnt, [fi], ones16, mask=mask)
        plsc.addupdate_scatter(acc_conf, [fi], conf, mask=mask)
        return gidx + _LANES

    lax.fori_loop(0, _ITERS, body, base + lane)
    pltpu.sync_copy(acc_cnt, cnt_out.at[wid])
    pltpu.sync_copy(acc_conf, sconf_out.at[wid])


def _combine_body(cnt_ref, sconf_ref, mbin_ref, macc_ref, ece_ref, ys_ref):
    cnt_tot = jnp.sum(cnt_ref[...], axis=0, keepdims=True)
    sconf_tot = jnp.sum(sconf_ref[...], axis=0, keepdims=True)
    mbin = mbin_ref[...]
    macc = macc_ref[...]
    count = jnp.dot(cnt_tot, mbin, preferred_element_type=jnp.float32)
    sum_acc = jnp.dot(cnt_tot, macc, preferred_element_type=jnp.float32)
    sum_conf = jnp.dot(sconf_tot, mbin, preferred_element_type=jnp.float32)
    lane32 = lax.broadcasted_iota(jnp.int32, (1, 32), 1)
    valid = (lane32 < _N_BINS) & (count > 0.0)
    safe = jnp.maximum(count, 1.0)
    acc_in = jnp.where(valid, sum_acc / safe, 0.0)
    conf_in = jnp.where(valid, sum_conf / safe, 0.0)
    prop = count / float(_N)
    per_bin = jnp.where(valid, jnp.abs(conf_in - acc_in) * prop, 0.0)
    ece_ref[...] = jnp.sum(per_bin, keepdims=True).reshape(1, 1)
    ys_ref[...] = acc_in


def kernel(logits, labels):
    n, c = logits.shape
    b = _ROW_BLOCK

    v_p = pl.pallas_call(
        _dense_body,
        grid=(_GRID,),
        in_specs=[pl.BlockSpec((b, c), lambda i: (i, 0))],
        out_specs=pl.BlockSpec((b // 8, 8), lambda i: (i, 0)),
        out_shape=jax.ShapeDtypeStruct((_NPAD // 8, 8), jnp.float32),
    )(logits)

    mesh = plsc.VectorSubcoreMesh(
        core_axis_name="c", subcore_axis_name="s",
        num_cores=2, num_subcores=16,
    )
    sc_hist = pl.kernel(
        _sc_hist_body,
        out_type=[
            jax.ShapeDtypeStruct((_NW, _KEYS * _LANES), jnp.float32),
            jax.ShapeDtypeStruct((_NW, _KEYS * _LANES), jnp.float32),
        ],
        mesh=mesh,
        compiler_params=pltpu.CompilerParams(needs_layout_passes=False),
        scratch_types=[
            pltpu.VMEM((_CHUNK,), jnp.float32),
            pltpu.VMEM((_CHUNK,), jnp.int32),
            pltpu.VMEM((_KEYS * _LANES,), jnp.float32),
            pltpu.VMEM((_KEYS * _LANES,), jnp.float32),
        ],
    )
    cnt_part, sconf_part = sc_hist(v_p.reshape(_NPAD), labels)

    flat_key = jnp.arange(_KEYS * _LANES, dtype=jnp.int32) // _LANES
    bins = jnp.arange(32, dtype=jnp.int32)
    mbin = ((flat_key % 32)[:, None] == bins[None, :]).astype(jnp.float32)
    macc = (flat_key[:, None] == (bins[None, :] + 32)).astype(jnp.float32)

    ece2, ys2 = pl.pallas_call(
        _combine_body,
        in_specs=[
            pl.BlockSpec((_NW, _KEYS * _LANES), lambda: (0, 0)),
            pl.BlockSpec((_NW, _KEYS * _LANES), lambda: (0, 0)),
            pl.BlockSpec((_KEYS * _LANES, 32), lambda: (0, 0)),
            pl.BlockSpec((_KEYS * _LANES, 32), lambda: (0, 0)),
        ],
        out_specs=[
            pl.BlockSpec((1, 1), lambda: (0, 0)),
            pl.BlockSpec((1, 32), lambda: (0, 0)),
        ],
        out_shape=[
            jax.ShapeDtypeStruct((1, 1), jnp.float32),
            jax.ShapeDtypeStruct((1, 32), jnp.float32),
        ],
    )(cnt_part, sconf_part, mbin, macc)

    return (ece2.reshape(1), ys2[0, :_N_BINS])


# speedup vs baseline: 2.8536x; 1.0532x over previous
import jax
import jax.numpy as jnp
from jax import lax
from jax.experimental import pallas as pl
from jax.experimental.pallas import tpu as pltpu
from jax.experimental.pallas import tpu_sc as plsc

_N = 1_000_000
_C = 100
_N_BINS = 20
_ROW_BLOCK = 8192
_GRID = 123
_NPAD = _GRID * _ROW_BLOCK
_NW = 32
_LANES = 16
_CHUNK = _NPAD // _NW
_ITERS = _CHUNK // _LANES
_KEYS = 64
_LAB_LAST = _N - (_NW - 1) * _CHUNK


def _dense_body(logits_ref, v_ref):
    x = logits_ref[...]
    col = lax.broadcasted_iota(jnp.int32, x.shape, 1)
    xp = lax.bitcast_convert_type(
        (lax.bitcast_convert_type(x, jnp.int32) & ~0x7F) | col, jnp.float32)
    mm = jnp.max(xp, axis=1, keepdims=True)
    s = jnp.sum(jnp.exp(xp - mm), axis=1, keepdims=True)
    si = lax.bitcast_convert_type(s, jnp.int32)
    mi = lax.bitcast_convert_type(mm, jnp.int32)
    v = lax.bitcast_convert_type((si & ~0x7F) | (mi & 0x7F), jnp.float32)
    v_ref[...] = v.reshape(_ROW_BLOCK // 32, 32)


def _sc_hist_body(v_hbm, labels_hbm, cnt_out, sconf_out,
                  v_v, lab_v, acc_cnt, acc_conf):
    wid = lax.axis_index("s") * 2 + lax.axis_index("c")
    base = wid * _CHUNK
    pltpu.sync_copy(v_hbm.at[pl.ds(base, _CHUNK)], v_v)

    @pl.when(wid < _NW - 1)
    def _():
        pltpu.sync_copy(labels_hbm.at[pl.ds(base, _CHUNK)], lab_v)

    @pl.when(wid == _NW - 1)
    def _():
        pltpu.sync_copy(labels_hbm.at[pl.ds(base, _LAB_LAST)],
                        lab_v.at[pl.ds(0, _LAB_LAST)])

    zeros16 = jnp.zeros((_LANES,), jnp.float32)
    for k in range(_KEYS):
        acc_cnt[pl.ds(k * _LANES, _LANES)] = zeros16
        acc_conf[pl.ds(k * _LANES, _LANES)] = zeros16
    lane = lax.iota(jnp.int32, _LANES)
    ones16 = jnp.ones((_LANES,), jnp.float32)

    def body(i, gidx):
        v16 = v_v[pl.ds(i * _LANES, _LANES)]
        lab16 = lab_v[pl.ds(i * _LANES, _LANES)]
        pred = lax.bitcast_convert_type(v16, jnp.int32) & 0x7F
        conf = 1.0 / v16
        bin_ = lax.convert_element_type(conf * float(_N_BINS), jnp.int32)
        bin_ = jnp.minimum(jnp.maximum(bin_, 0), _N_BINS - 1)
        fi = jnp.where(pred == lab16, 32 * _LANES, 0) + bin_ * _LANES + lane
        mask = gidx < _N
        plsc.addupdate_scatter(acc_cnt, [fi], ones16, mask=mask)
        plsc.addupdate_scatter(acc_conf, [fi], conf, mask=mask)
        return gidx + _LANES

    lax.fori_loop(0, _ITERS, body, base + lane)
    pltpu.sync_copy(acc_cnt, cnt_out.at[wid])
    pltpu.sync_copy(acc_conf, sconf_out.at[wid])


def _combine_body(cnt_ref, sconf_ref, mbin_ref, macc_ref, ece_ref, ys_ref):
    cnt_tot = jnp.sum(cnt_ref[...], axis=0, keepdims=True)
    sconf_tot = jnp.sum(sconf_ref[...], axis=0, keepdims=True)
    mbin = mbin_ref[...]
    macc = macc_ref[...]
    count = jnp.dot(cnt_tot, mbin, preferred_element_type=jnp.float32)
    sum_acc = jnp.dot(cnt_tot, macc, preferred_element_type=jnp.float32)
    sum_conf = jnp.dot(sconf_tot, mbin, preferred_element_type=jnp.float32)
    lane32 = lax.broadcasted_iota(jnp.int32, (1, 32), 1)
    valid = (lane32 < _N_BINS) & (count > 0.0)
    safe = jnp.maximum(count, 1.0)
    acc_in = jnp.where(valid, sum_acc / safe, 0.0)
    conf_in = jnp.where(valid, sum_conf / safe, 0.0)
    prop = count / float(_N)
    per_bin = jnp.where(valid, jnp.abs(conf_in - acc_in) * prop, 0.0)
    ece_ref[...] = jnp.sum(per_bin, keepdims=True).reshape(1, 1)
    ys_ref[...] = acc_in


def kernel(logits, labels):
    n, c = logits.shape
    b = _ROW_BLOCK

    v_p = pl.pallas_call(
        _dense_body,
        grid=(_GRID,),
        in_specs=[pl.BlockSpec((b, c), lambda i: (i, 0))],
        out_specs=pl.BlockSpec((b // 32, 32), lambda i: (i, 0)),
        out_shape=jax.ShapeDtypeStruct((_NPAD // 32, 32), jnp.float32),
    )(logits)

    mesh = plsc.VectorSubcoreMesh(
        core_axis_name="c", subcore_axis_name="s",
        num_cores=2, num_subcores=16,
    )
    sc_hist = pl.kernel(
        _sc_hist_body,
        out_type=[
            jax.ShapeDtypeStruct((_NW, _KEYS * _LANES), jnp.float32),
            jax.ShapeDtypeStruct((_NW, _KEYS * _LANES), jnp.float32),
        ],
        mesh=mesh,
        compiler_params=pltpu.CompilerParams(needs_layout_passes=False),
        scratch_types=[
            pltpu.VMEM((_CHUNK,), jnp.float32),
            pltpu.VMEM((_CHUNK,), jnp.int32),
            pltpu.VMEM((_KEYS * _LANES,), jnp.float32),
            pltpu.VMEM((_KEYS * _LANES,), jnp.float32),
        ],
    )
    cnt_part, sconf_part = sc_hist(v_p.reshape(_NPAD), labels)

    flat_key = jnp.arange(_KEYS * _LANES, dtype=jnp.int32) // _LANES
    bins = jnp.arange(32, dtype=jnp.int32)
    mbin = ((flat_key % 32)[:, None] == bins[None, :]).astype(jnp.float32)
    macc = (flat_key[:, None] == (bins[None, :] + 32)).astype(jnp.float32)

    ece2, ys2 = pl.pallas_call(
        _combine_body,
        in_specs=[
            pl.BlockSpec((_NW, _KEYS * _LANES), lambda: (0, 0)),
            pl.BlockSpec((_NW, _KEYS * _LANES), lambda: (0, 0)),
            pl.BlockSpec((_KEYS * _LANES, 32), lambda: (0, 0)),
            pl.BlockSpec((_KEYS * _LANES, 32), lambda: (0, 0)),
        ],
        out_specs=[
            pl.BlockSpec((1, 1), lambda: (0, 0)),
            pl.BlockSpec((1, 32), lambda: (0, 0)),
        ],
        out_shape=[
            jax.ShapeDtypeStruct((1, 1), jnp.float32),
            jax.ShapeDtypeStruct((1, 32), jnp.float32),
        ],
    )(cnt_part, sconf_part, mbin, macc)

    return (ece2.reshape(1), ys2[0, :_N_BINS])


# speedup vs baseline: 2.8655x; 1.0042x over previous
import jax
import jax.numpy as jnp
from jax import lax
from jax.experimental import pallas as pl
from jax.experimental.pallas import tpu as pltpu
from jax.experimental.pallas import tpu_sc as plsc

_N = 1_000_000
_C = 100
_N_BINS = 20
_ROW_BLOCK = 16384
_GRID = 62
_NPAD = _GRID * _ROW_BLOCK
_NW = 32
_LANES = 16
_CHUNK = _NPAD // _NW
_ITERS = _CHUNK // _LANES
_KEYS = 64
_LAB_LAST = _N - (_NW - 1) * _CHUNK


def _dense_body(logits_ref, v_ref):
    x = logits_ref[...]
    col = lax.broadcasted_iota(jnp.int32, x.shape, 1)
    xp = lax.bitcast_convert_type(
        (lax.bitcast_convert_type(x, jnp.int32) & ~0x7F) | col, jnp.float32)
    mm = jnp.max(xp, axis=1, keepdims=True)
    s = jnp.sum(jnp.exp(xp - mm), axis=1, keepdims=True)
    si = lax.bitcast_convert_type(s, jnp.int32)
    mi = lax.bitcast_convert_type(mm, jnp.int32)
    v = lax.bitcast_convert_type((si & ~0x7F) | (mi & 0x7F), jnp.float32)
    v_ref[...] = v.reshape(_ROW_BLOCK // 32, 32)


def _sc_hist_body(v_hbm, labels_hbm, cnt_out, sconf_out,
                  v_v, lab_v, acc_cnt, acc_conf):
    wid = lax.axis_index("s") * 2 + lax.axis_index("c")
    base = wid * _CHUNK
    pltpu.sync_copy(v_hbm.at[pl.ds(base, _CHUNK)], v_v)

    @pl.when(wid < _NW - 1)
    def _():
        pltpu.sync_copy(labels_hbm.at[pl.ds(base, _CHUNK)], lab_v)

    @pl.when(wid == _NW - 1)
    def _():
        pltpu.sync_copy(labels_hbm.at[pl.ds(base, _LAB_LAST)],
                        lab_v.at[pl.ds(0, _LAB_LAST)])

    zeros16 = jnp.zeros((_LANES,), jnp.float32)
    for k in range(_KEYS):
        acc_cnt[pl.ds(k * _LANES, _LANES)] = zeros16
        acc_conf[pl.ds(k * _LANES, _LANES)] = zeros16
    lane = lax.iota(jnp.int32, _LANES)
    ones16 = jnp.ones((_LANES,), jnp.float32)

    def body(i, gidx):
        v16 = v_v[pl.ds(i * _LANES, _LANES)]
        lab16 = lab_v[pl.ds(i * _LANES, _LANES)]
        pred = lax.bitcast_convert_type(v16, jnp.int32) & 0x7F
        conf = 1.0 / v16
        bin_ = lax.convert_element_type(conf * float(_N_BINS), jnp.int32)
        bin_ = jnp.minimum(jnp.maximum(bin_, 0), _N_BINS - 1)
        fi = jnp.where(pred == lab16, 32 * _LANES, 0) + bin_ * _LANES + lane
        mask = gidx < _N
        plsc.addupdate_scatter(acc_cnt, [fi], ones16, mask=mask)
        plsc.addupdate_scatter(acc_conf, [fi], conf, mask=mask)
        return gidx + _LANES

    lax.fori_loop(0, _ITERS, body, base + lane)
    pltpu.sync_copy(acc_cnt, cnt_out.at[wid])
    pltpu.sync_copy(acc_conf, sconf_out.at[wid])


def _combine_body(cnt_ref, sconf_ref, mbin_ref, macc_ref, ece_ref, ys_ref):
    cnt_tot = jnp.sum(cnt_ref[...], axis=0, keepdims=True)
    sconf_tot = jnp.sum(sconf_ref[...], axis=0, keepdims=True)
    mbin = mbin_ref[...]
    macc = macc_ref[...]
    count = jnp.dot(cnt_tot, mbin, preferred_element_type=jnp.float32)
    sum_acc = jnp.dot(cnt_tot, macc, preferred_element_type=jnp.float32)
    sum_conf = jnp.dot(sconf_tot, mbin, preferred_element_type=jnp.float32)
    lane32 = lax.broadcasted_iota(jnp.int32, (1, 32), 1)
    valid = (lane32 < _N_BINS) & (count > 0.0)
    safe = jnp.maximum(count, 1.0)
    acc_in = jnp.where(valid, sum_acc / safe, 0.0)
    conf_in = jnp.where(valid, sum_conf / safe, 0.0)
    prop = count / float(_N)
    per_bin = jnp.where(valid, jnp.abs(conf_in - acc_in) * prop, 0.0)
    ece_ref[...] = jnp.sum(per_bin, keepdims=True).reshape(1, 1)
    ys_ref[...] = acc_in


def kernel(logits, labels):
    n, c = logits.shape
    b = _ROW_BLOCK

    v_p = pl.pallas_call(
        _dense_body,
        grid=(_GRID,),
        in_specs=[pl.BlockSpec((b, c), lambda i: (i, 0))],
        out_specs=pl.BlockSpec((b // 32, 32), lambda i: (i, 0)),
        out_shape=jax.ShapeDtypeStruct((_NPAD // 32, 32), jnp.float32),
    )(logits)

    mesh = plsc.VectorSubcoreMesh(
        core_axis_name="c", subcore_axis_name="s",
        num_cores=2, num_subcores=16,
    )
    sc_hist = pl.kernel(
        _sc_hist_body,
        out_type=[
            jax.ShapeDtypeStruct((_NW, _KEYS * _LANES), jnp.float32),
            jax.ShapeDtypeStruct((_NW, _KEYS * _LANES), jnp.float32),
        ],
        mesh=mesh,
        compiler_params=pltpu.CompilerParams(needs_layout_passes=False),
        scratch_types=[
            pltpu.VMEM((_CHUNK,), jnp.float32),
            pltpu.VMEM((_CHUNK,), jnp.int32),
            pltpu.VMEM((_KEYS * _LANES,), jnp.float32),
            pltpu.VMEM((_KEYS * _LANES,), jnp.float32),
        ],
    )
    cnt_part, sconf_part = sc_hist(v_p.reshape(_NPAD), labels)

    flat_key = jnp.arange(_KEYS * _LANES, dtype=jnp.int32) // _LANES
    bins = jnp.arange(32, dtype=jnp.int32)
    mbin = ((flat_key % 32)[:, None] == bins[None, :]).astype(jnp.float32)
    macc = (flat_key[:, None] == (bins[None, :] + 32)).astype(jnp.float32)

    ece2, ys2 = pl.pallas_call(
        _combine_body,
        in_specs=[
            pl.BlockSpec((_NW, _KEYS * _LANES), lambda: (0, 0)),
            pl.BlockSpec((_NW, _KEYS * _LANES), lambda: (0, 0)),
            pl.BlockSpec((_KEYS * _LANES, 32), lambda: (0, 0)),
            pl.BlockSpec((_KEYS * _LANES, 32), lambda: (0, 0)),
        ],
        out_specs=[
            pl.BlockSpec((1, 1), lambda: (0, 0)),
            pl.BlockSpec((1, 32), lambda: (0, 0)),
        ],
        out_shape=[
            jax.ShapeDtypeStruct((1, 1), jnp.float32),
            jax.ShapeDtypeStruct((1, 32), jnp.float32),
        ],
    )(cnt_part, sconf_part, mbin, macc)

    return (ece2.reshape(1), ys2[0, :_N_BINS])


# speedup vs baseline: 2.8682x; 1.0009x over previous
import jax
import jax.numpy as jnp
from jax import lax
from jax.experimental import pallas as pl
from jax.experimental.pallas import tpu as pltpu
from jax.experimental.pallas import tpu_sc as plsc

_N = 1_000_000
_C = 100
_N_BINS = 20
_ROW_BLOCK = 16384
_GRID = 62
_NPAD = _GRID * _ROW_BLOCK
_NW = 32
_LANES = 16
_CHUNK = _NPAD // _NW
_ITERS = _CHUNK // _LANES
_KEYS = 64
_LAB_LAST = _N - (_NW - 1) * _CHUNK


def _dense_body(logits_ref, v_ref):
    x = logits_ref[...]
    col = lax.broadcasted_iota(jnp.int32, x.shape, 1)
    xp = lax.bitcast_convert_type(
        (lax.bitcast_convert_type(x, jnp.int32) & ~0x7F) | col, jnp.float32)
    mm = jnp.max(xp, axis=1, keepdims=True)
    s = jnp.sum(jnp.exp(x - mm), axis=1, keepdims=True)
    si = lax.bitcast_convert_type(s, jnp.int32)
    mi = lax.bitcast_convert_type(mm, jnp.int32)
    v = lax.bitcast_convert_type((si & ~0x7F) | (mi & 0x7F), jnp.float32)
    v_ref[...] = v.reshape(_ROW_BLOCK // 32, 32)


def _sc_hist_body(v_hbm, labels_hbm, cnt_out, sconf_out,
                  v_v, lab_v, acc_cnt, acc_conf):
    wid = lax.axis_index("s") * 2 + lax.axis_index("c")
    base = wid * _CHUNK
    pltpu.sync_copy(v_hbm.at[pl.ds(base, _CHUNK)], v_v)

    @pl.when(wid < _NW - 1)
    def _():
        pltpu.sync_copy(labels_hbm.at[pl.ds(base, _CHUNK)], lab_v)

    @pl.when(wid == _NW - 1)
    def _():
        pltpu.sync_copy(labels_hbm.at[pl.ds(base, _LAB_LAST)],
                        lab_v.at[pl.ds(0, _LAB_LAST)])

    zeros16 = jnp.zeros((_LANES,), jnp.float32)
    for k in range(_KEYS):
        acc_cnt[pl.ds(k * _LANES, _LANES)] = zeros16
        acc_conf[pl.ds(k * _LANES, _LANES)] = zeros16
    lane = lax.iota(jnp.int32, _LANES)
    ones16 = jnp.ones((_LANES,), jnp.float32)

    def body(i, gidx):
        v16 = v_v[pl.ds(i * _LANES, _LANES)]
        lab16 = lab_v[pl.ds(i * _LANES, _LANES)]
        pred = lax.bitcast_convert_type(v16, jnp.int32) & 0x7F
        conf = 1.0 / v16
        bin_ = lax.convert_element_type(conf * float(_N_BINS), jnp.int32)
        bin_ = jnp.minimum(jnp.maximum(bin_, 0), _N_BINS - 1)
        fi = jnp.where(pred == lab16, 32 * _LANES, 0) + bin_ * _LANES + lane
        mask = gidx < _N
        plsc.addupdate_scatter(acc_cnt, [fi], ones16, mask=mask)
        plsc.addupdate_scatter(acc_conf, [fi], conf, mask=mask)
        return gidx + _LANES

    lax.fori_loop(0, _ITERS, body, base + lane)
    pltpu.sync_copy(acc_cnt, cnt_out.at[wid])
    pltpu.sync_copy(acc_conf, sconf_out.at[wid])


def _combine_body(cnt_ref, sconf_ref, mbin_ref, macc_ref, ece_ref, ys_ref):
    cnt_tot = jnp.sum(cnt_ref[...], axis=0, keepdims=True)
    sconf_tot = jnp.sum(sconf_ref[...], axis=0, keepdims=True)
    mbin = mbin_ref[...]
    macc = macc_ref[...]
    count = jnp.dot(cnt_tot, mbin, preferred_element_type=jnp.float32)
    sum_acc = jnp.dot(cnt_tot, macc, preferred_element_type=jnp.float32)
    sum_conf = jnp.dot(sconf_tot, mbin, preferred_element_type=jnp.float32)
    lane32 = lax.broadcasted_iota(jnp.int32, (1, 32), 1)
    valid = (lane32 < _N_BINS) & (count > 0.0)
    safe = jnp.maximum(count, 1.0)
    acc_in = jnp.where(valid, sum_acc / safe, 0.0)
    conf_in = jnp.where(valid, sum_conf / safe, 0.0)
    prop = count / float(_N)
    per_bin = jnp.where(valid, jnp.abs(conf_in - acc_in) * prop, 0.0)
    ece_ref[...] = jnp.sum(per_bin, keepdims=True).reshape(1, 1)
    ys_ref[...] = acc_in


def kernel(logits, labels):
    n, c = logits.shape
    b = _ROW_BLOCK

    v_p = pl.pallas_call(
        _dense_body,
        grid=(_GRID,),
        in_specs=[pl.BlockSpec((b, c), lambda i: (i, 0))],
        out_specs=pl.BlockSpec((b // 32, 32), lambda i: (i, 0)),
        out_shape=jax.ShapeDtypeStruct((_NPAD // 32, 32), jnp.float32),
    )(logits)

    mesh = plsc.VectorSubcoreMesh(
        core_axis_name="c", subcore_axis_name="s",
        num_cores=2, num_subcores=16,
    )
    sc_hist = pl.kernel(
        _sc_hist_body,
        out_type=[
            jax.ShapeDtypeStruct((_NW, _KEYS * _LANES), jnp.float32),
            jax.ShapeDtypeStruct((_NW, _KEYS * _LANES), jnp.float32),
        ],
        mesh=mesh,
        compiler_params=pltpu.CompilerParams(needs_layout_passes=False),
        scratch_types=[
            pltpu.VMEM((_CHUNK,), jnp.float32),
            pltpu.VMEM((_CHUNK,), jnp.int32),
            pltpu.VMEM((_KEYS * _LANES,), jnp.float32),
            pltpu.VMEM((_KEYS * _LANES,), jnp.float32),
        ],
    )
    cnt_part, sconf_part = sc_hist(v_p.reshape(_NPAD), labels)

    flat_key = jnp.arange(_KEYS * _LANES, dtype=jnp.int32) // _LANES
    bins = jnp.arange(32, dtype=jnp.int32)
    mbin = ((flat_key % 32)[:, None] == bins[None, :]).astype(jnp.float32)
    macc = (flat_key[:, None] == (bins[None, :] + 32)).astype(jnp.float32)

    ece2, ys2 = pl.pallas_call(
        _combine_body,
        in_specs=[
            pl.BlockSpec((_NW, _KEYS * _LANES), lambda: (0, 0)),
            pl.BlockSpec((_NW, _KEYS * _LANES), lambda: (0, 0)),
            pl.BlockSpec((_KEYS * _LANES, 32), lambda: (0, 0)),
            pl.BlockSpec((_KEYS * _LANES, 32), lambda: (0, 0)),
        ],
        out_specs=[
            pl.BlockSpec((1, 1), lambda: (0, 0)),
            pl.BlockSpec((1, 32), lambda: (0, 0)),
        ],
        out_shape=[
            jax.ShapeDtypeStruct((1, 1), jnp.float32),
            jax.ShapeDtypeStruct((1, 32), jnp.float32),
        ],
    )(cnt_part, sconf_part, mbin, macc)

    return (ece2.reshape(1), ys2[0, :_N_BINS])


# speedup vs baseline: 2.8840x; 1.0055x over previous
import jax
import jax.numpy as jnp
from jax import lax
from jax.experimental import pallas as pl
from jax.experimental.pallas import tpu as pltpu
from jax.experimental.pallas import tpu_sc as plsc

_N = 1_000_000
_C = 100
_N_BINS = 20
_ROW_BLOCK = 16384
_GRID = 62
_NPAD = _GRID * _ROW_BLOCK
_NW = 32
_LANES = 16
_CHUNK = _NPAD // _NW
_ITERS = _CHUNK // _LANES
_KEYS = 64
_LAB_LAST = _N - (_NW - 1) * _CHUNK


def _dense_body(logits_ref, v_ref):
    x = logits_ref[...]
    col = lax.broadcasted_iota(jnp.int32, x.shape, 1)
    xp = lax.bitcast_convert_type(
        (lax.bitcast_convert_type(x, jnp.int32) & ~0x7F) | col, jnp.float32)
    mm = jnp.max(xp, axis=1, keepdims=True)
    s = jnp.sum(jnp.exp(x - mm), axis=1, keepdims=True)
    si = lax.bitcast_convert_type(s, jnp.int32)
    mi = lax.bitcast_convert_type(mm, jnp.int32)
    v = lax.bitcast_convert_type((si & ~0x7F) | (mi & 0x7F), jnp.float32)
    v_ref[...] = v.reshape(_ROW_BLOCK // 64, 64)


def _sc_hist_body(v_hbm, labels_hbm, cnt_out, sconf_out,
                  v_v, lab_v, acc_cnt, acc_conf):
    wid = lax.axis_index("s") * 2 + lax.axis_index("c")
    base = wid * _CHUNK
    pltpu.sync_copy(v_hbm.at[pl.ds(base, _CHUNK)], v_v)

    @pl.when(wid < _NW - 1)
    def _():
        pltpu.sync_copy(labels_hbm.at[pl.ds(base, _CHUNK)], lab_v)

    @pl.when(wid == _NW - 1)
    def _():
        pltpu.sync_copy(labels_hbm.at[pl.ds(base, _LAB_LAST)],
                        lab_v.at[pl.ds(0, _LAB_LAST)])

    zeros16 = jnp.zeros((_LANES,), jnp.float32)
    for k in range(_KEYS):
        acc_cnt[pl.ds(k * _LANES, _LANES)] = zeros16
        acc_conf[pl.ds(k * _LANES, _LANES)] = zeros16
    lane = lax.iota(jnp.int32, _LANES)
    ones16 = jnp.ones((_LANES,), jnp.float32)

    def body(i, gidx):
        v16 = v_v[pl.ds(i * _LANES, _LANES)]
        lab16 = lab_v[pl.ds(i * _LANES, _LANES)]
        pred = lax.bitcast_convert_type(v16, jnp.int32) & 0x7F
        conf = 1.0 / v16
        bin_ = lax.convert_element_type(conf * float(_N_BINS), jnp.int32)
        bin_ = jnp.minimum(jnp.maximum(bin_, 0), _N_BINS - 1)
        fi = jnp.where(pred == lab16, 32 * _LANES, 0) + bin_ * _LANES + lane
        mask = gidx < _N
        plsc.addupdate_scatter(acc_cnt, [fi], ones16, mask=mask)
        plsc.addupdate_scatter(acc_conf, [fi], conf, mask=mask)
        return gidx + _LANES

    lax.fori_loop(0, _ITERS, body, base + lane)
    pltpu.sync_copy(acc_cnt, cnt_out.at[wid])
    pltpu.sync_copy(acc_conf, sconf_out.at[wid])


def _combine_body(cnt_ref, sconf_ref, mbin_ref, macc_ref, ece_ref, ys_ref):
    cnt_tot = jnp.sum(cnt_ref[...], axis=0, keepdims=True)
    sconf_tot = jnp.sum(sconf_ref[...], axis=0, keepdims=True)
    mbin = mbin_ref[...]
    macc = macc_ref[...]
    count = jnp.dot(cnt_tot, mbin, preferred_element_type=jnp.float32)
    sum_acc = jnp.dot(cnt_tot, macc, preferred_element_type=jnp.float32)
    sum_conf = jnp.dot(sconf_tot, mbin, preferred_element_type=jnp.float32)
    lane32 = lax.broadcasted_iota(jnp.int32, (1, 32), 1)
    valid = (lane32 < _N_BINS) & (count > 0.0)
    safe = jnp.maximum(count, 1.0)
    acc_in = jnp.where(valid, sum_acc / safe, 0.0)
    conf_in = jnp.where(valid, sum_conf / safe, 0.0)
    prop = count / float(_N)
    per_bin = jnp.where(valid, jnp.abs(conf_in - acc_in) * prop, 0.0)
    ece_ref[...] = jnp.sum(per_bin, keepdims=True).reshape(1, 1)
    ys_ref[...] = acc_in


def kernel(logits, labels):
    n, c = logits.shape
    b = _ROW_BLOCK

    v_p = pl.pallas_call(
        _dense_body,
        grid=(_GRID,),
        in_specs=[pl.BlockSpec((b, c), lambda i: (i, 0))],
        out_specs=pl.BlockSpec((b // 64, 64), lambda i: (i, 0)),
        out_shape=jax.ShapeDtypeStruct((_NPAD // 64, 64), jnp.float32),
    )(logits)

    mesh = plsc.VectorSubcoreMesh(
        core_axis_name="c", subcore_axis_name="s",
        num_cores=2, num_subcores=16,
    )
    sc_hist = pl.kernel(
        _sc_hist_body,
        out_type=[
            jax.ShapeDtypeStruct((_NW, _KEYS * _LANES), jnp.float32),
            jax.ShapeDtypeStruct((_NW, _KEYS * _LANES), jnp.float32),
        ],
        mesh=mesh,
        compiler_params=pltpu.CompilerParams(needs_layout_passes=False),
        scratch_types=[
            pltpu.VMEM((_CHUNK,), jnp.float32),
            pltpu.VMEM((_CHUNK,), jnp.int32),
            pltpu.VMEM((_KEYS * _LANES,), jnp.float32),
            pltpu.VMEM((_KEYS * _LANES,), jnp.float32),
        ],
    )
    cnt_part, sconf_part = sc_hist(v_p.reshape(_NPAD), labels)

    flat_key = jnp.arange(_KEYS * _LANES, dtype=jnp.int32) // _LANES
    bins = jnp.arange(32, dtype=jnp.int32)
    mbin = ((flat_key % 32)[:, None] == bins[None, :]).astype(jnp.float32)
    macc = (flat_key[:, None] == (bins[None, :] + 32)).astype(jnp.float32)

    ece2, ys2 = pl.pallas_call(
        _combine_body,
        in_specs=[
            pl.BlockSpec((_NW, _KEYS * _LANES), lambda: (0, 0)),
            pl.BlockSpec((_NW, _KEYS * _LANES), lambda: (0, 0)),
            pl.BlockSpec((_KEYS * _LANES, 32), lambda: (0, 0)),
            pl.BlockSpec((_KEYS * _LANES, 32), lambda: (0, 0)),
        ],
        out_specs=[
            pl.BlockSpec((1, 1), lambda: (0, 0)),
            pl.BlockSpec((1, 32), lambda: (0, 0)),
        ],
        out_shape=[
            jax.ShapeDtypeStruct((1, 1), jnp.float32),
            jax.ShapeDtypeStruct((1, 32), jnp.float32),
        ],
    )(cnt_part, sconf_part, mbin, macc)

    return (ece2.reshape(1), ys2[0, :_N_BINS])
